# Initial kernel scaffold; baseline (speedup 1.0000x reference)
import jax
import jax.numpy as jnp
from jax.experimental import pallas as pl


def kernel(grid, guidemap):
    raise NotImplementedError("write your pallas kernel here")


# speedup vs baseline: 1620.0977x; 1620.0977x over previous
import functools

import jax
import jax.numpy as jnp
from jax.experimental import pallas as pl

_N, _C, _D, _GH, _GW = 4, 12, 8, 16, 16
_H = 512
_W = 512
_HB = 128


def _tent_matrix(n_out, n_in):
    coord = jnp.arange(n_out, dtype=jnp.float32) / (n_out - 1) * 2.0 - 1.0
    i = jnp.clip((coord + 1.0) * 0.5 * (n_in - 1), 0.0, float(n_in - 1))
    k = jnp.arange(n_in, dtype=jnp.float32)
    return jnp.maximum(0.0, 1.0 - jnp.abs(i[:, None] - k[None, :]))


def _slice_kernel(grid_ref, by_ref, bxt_ref, gm_ref, out_ref):
    by = by_ref[...]
    g = grid_ref[0]
    tmp = jnp.dot(by, g, preferred_element_type=jnp.float32)
    gm = gm_ref[0]
    iz = jnp.clip(((gm * 2.0 - 1.0) + 1.0) * 0.5 * (_D - 1), 0.0, float(_D - 1))
    bxt = bxt_ref[...]
    wzs = [jnp.maximum(0.0, 1.0 - jnp.abs(iz - z)) for z in range(_D)]
    for c in range(_C):
        acc = jnp.zeros((_HB, _W), dtype=jnp.float32)
        for z in range(_D):
            sl = tmp[:, (c * _D + z) * _GW:(c * _D + z + 1) * _GW]
            gup = jnp.dot(sl, bxt, preferred_element_type=jnp.float32)
            acc = acc + gup * wzs[z]
        out_ref[0, c] = acc


@functools.partial(jax.jit, static_argnames=())
def kernel(grid, guidemap):
    grid_r = grid.reshape(_N, _C * _D, _GH, _GW).transpose(0, 2, 1, 3)
    grid_r = grid_r.reshape(_N, _GH, _C * _D * _GW)
    by = _tent_matrix(_H, _GH)
    bxt = _tent_matrix(_W, _GW).T
    gm = guidemap.reshape(_N, _H, _W)

    nh = _H // _HB
    out = pl.pallas_call(
        _slice_kernel,
        grid=(_N, nh),
        in_specs=[
            pl.BlockSpec((1, _GH, _C * _D * _GW), lambda n, h: (n, 0, 0)),
            pl.BlockSpec((_HB, _GH), lambda n, h: (h, 0)),
            pl.BlockSpec((_GW, _W), lambda n, h: (0, 0)),
            pl.BlockSpec((1, _HB, _W), lambda n, h: (n, h, 0)),
        ],
        out_specs=pl.BlockSpec((1, _C, _HB, _W), lambda n, h: (n, 0, h, 0)),
        out_shape=jax.ShapeDtypeStruct((_N, _C, _H, _W), jnp.float32),
    )(grid_r, by, bxt, gm)
    return out
